# baseline (speedup 1.0000x reference)
import jax
import jax.numpy as jnp
from jax import lax
from jax.experimental import pallas as pl
from jax.experimental.pallas import tpu as pltpu
from jax.experimental.pallas import tpu_sc as plsc

BATCH = 16384
EMB_D = 64
NC = 2
NS = 16
NW = NC * NS
BPW = BATCH // NW
CHUNK = 128
NCHUNK = BPW // CHUNK
NGRP = BPW // 16
SPLIT = 499968


def _sc_body(eu, ei1, ei2, lw, ubw, ibw, uidx, iidx,
             bd_o, lu_o, ub_o, ib_o,
             idx_u, idx_i, idx_i1, idx_i2, bidx_u, bidx_i,
             urows, irows1, irows2, ublk_l, ublk_b, iblk_b,
             luv, ubv, ibv, bdv, sem):
    c = lax.axis_index("c")
    s = lax.axis_index("s")
    wid = s * NC + c
    base = wid * BPW
    rbase = wid * NCHUNK
    pltpu.sync_copy(uidx.at[pl.ds(rbase, NCHUNK)], idx_u)
    pltpu.sync_copy(iidx.at[pl.ds(rbase, NCHUNK)], idx_i)
    for j in range(NCHUNK):
        for k in range(CHUNK // 16):
            sl = pl.ds(k * 16, 16)
            v_u = idx_u[j, sl]
            v_i = idx_i[j, sl]
            bidx_u[j, sl] = lax.shift_right_logical(v_u, 4)
            bidx_i[j, sl] = lax.shift_right_logical(v_i, 4)
            idx_i1[j, sl] = jnp.minimum(v_i, SPLIT - 1)
            idx_i2[j, sl] = jnp.maximum(v_i - SPLIT, 0)
    cps = []
    for j in range(NCHUNK):
        dst = pl.ds(j * CHUNK, CHUNK)
        cps.append(pltpu.async_copy(eu.at[idx_u.at[j]], urows.at[dst], sem))
        cps.append(pltpu.async_copy(ei1.at[idx_i1.at[j]], irows1.at[dst], sem))
        cps.append(pltpu.async_copy(ei2.at[idx_i2.at[j]], irows2.at[dst], sem))
        cps.append(pltpu.async_copy(lw.at[bidx_u.at[j]], ublk_l.at[dst], sem))
        cps.append(pltpu.async_copy(ubw.at[bidx_u.at[j]], ublk_b.at[dst], sem))
        cps.append(pltpu.async_copy(ibw.at[bidx_i.at[j]], iblk_b.at[dst], sem))
    for cp in cps:
        cp.wait()
    iota = lax.iota(jnp.int32, 16)
    fifteen = jnp.full((16,), 15, jnp.int32)
    for g in range(NGRP):
        rows = iota + (g * 16)
        sl = pl.ds((g % 8) * 16, 16)
        v_i = idx_i[g // 8, sl]
        offs_u = jnp.bitwise_and(idx_u[g // 8, sl], fifteen)
        offs_i = jnp.bitwise_and(v_i, fifteen)
        in_lo = v_i < SPLIT
        out_sl = pl.ds(g * 16, 16)
        luv[out_sl] = plsc.load_gather(ublk_l, [rows, offs_u])
        ubv[out_sl] = plsc.load_gather(ublk_b, [rows, offs_u])
        ibv[out_sl] = plsc.load_gather(iblk_b, [rows, offs_i])

        def dcol(dd, acc, rows=rows, in_lo=in_lo):
            cols = jnp.broadcast_to(dd, (16,))
            u = plsc.load_gather(urows, [rows, cols])
            i1 = plsc.load_gather(irows1, [rows, cols])
            i2 = plsc.load_gather(irows2, [rows, cols])
            it = jnp.where(in_lo, i1, i2)
            return acc + jnp.abs(it - u)

        acc = lax.fori_loop(0, EMB_D, dcol, jnp.zeros((16,), jnp.float32))
        bdv[out_sl] = acc
    pltpu.sync_copy(bdv, bd_o.at[pl.ds(base, BPW)])
    pltpu.sync_copy(luv, lu_o.at[pl.ds(base, BPW)])
    pltpu.sync_copy(ubv, ub_o.at[pl.ds(base, BPW)])
    pltpu.sync_copy(ibv, ib_o.at[pl.ds(base, BPW)])


def _sc_gather(uidx2d, iidx2d, eu, ei1, ei2, lw16, ubw16, ibw16):
    mesh = plsc.VectorSubcoreMesh(core_axis_name="c", subcore_axis_name="s")
    f = pl.kernel(
        _sc_body,
        out_type=[
            jax.ShapeDtypeStruct((BATCH,), jnp.float32),
            jax.ShapeDtypeStruct((BATCH,), jnp.float32),
            jax.ShapeDtypeStruct((BATCH,), jnp.float32),
            jax.ShapeDtypeStruct((BATCH,), jnp.float32),
        ],
        mesh=mesh,
        compiler_params=pltpu.CompilerParams(
            needs_layout_passes=False, use_tc_tiling_on_sc=False),
        scratch_types=[
            pltpu.VMEM((NCHUNK, CHUNK), jnp.int32),
            pltpu.VMEM((NCHUNK, CHUNK), jnp.int32),
            pltpu.VMEM((NCHUNK, CHUNK), jnp.int32),
            pltpu.VMEM((NCHUNK, CHUNK), jnp.int32),
            pltpu.VMEM((NCHUNK, CHUNK), jnp.int32),
            pltpu.VMEM((NCHUNK, CHUNK), jnp.int32),
            pltpu.VMEM((BPW, EMB_D), jnp.float32),
            pltpu.VMEM((BPW, EMB_D), jnp.float32),
            pltpu.VMEM((BPW, EMB_D), jnp.float32),
            pltpu.VMEM((BPW, 16), jnp.float32),
            pltpu.VMEM((BPW, 16), jnp.float32),
            pltpu.VMEM((BPW, 16), jnp.float32),
            pltpu.VMEM((BPW,), jnp.float32),
            pltpu.VMEM((BPW,), jnp.float32),
            pltpu.VMEM((BPW,), jnp.float32),
            pltpu.VMEM((BPW,), jnp.float32),
            pltpu.SemaphoreType.DMA,
        ],
    )
    return f(eu, ei1, ei2, lw16, ubw16, ibw16, uidx2d, iidx2d)


def _tc_decay_body(scal, r_ref, bl_ref):
    cc = scal[0]
    r = r_ref[...]
    mask = (r > 0.0).astype(jnp.float32)
    dt = r * mask + cc
    bl_ref[...] = jnp.sum(lax.rsqrt(dt) * mask, axis=1)


def _tc_final_body(scal, bd_ref, bl_ref, lu_ref, ub_ref, ib_ref, o_ref):
    gl = scal[1]
    alpha = scal[2]
    beta = scal[3]
    gamma = scal[4]
    bd = bd_ref[...]
    lamb = gl + jnp.clip(lu_ref[...], 0.1, 10.0)
    act = jnp.minimum(bl_ref[...] * lamb, bd)
    dist = bd - act
    logit = alpha * dist + beta * dist * dist + gamma + ub_ref[...] + ib_ref[...]
    o_ref[...] = 1.0 / (1.0 + jnp.exp(-logit))


def kernel(user_indices, item_indices, r_interval, emb_user, emb_item,
           user_lamb_w, user_bias_w, item_bias_w, global_lamb, alpha, beta,
           gamma, cutoff):
    uidx = user_indices.astype(jnp.int32).reshape(BATCH // 128, 128)
    iidx = item_indices.astype(jnp.int32).reshape(BATCH // 128, 128)
    cc = jnp.clip(cutoff, 0.1, 100.0)
    gl = jnp.clip(global_lamb, 0.01, 10.0)
    scal = jnp.stack([cc, gl, alpha, beta, gamma]).astype(jnp.float32)
    bl = pl.pallas_call(
        _tc_decay_body,
        out_shape=jax.ShapeDtypeStruct((BATCH,), jnp.float32),
        in_specs=[
            pl.BlockSpec(memory_space=pltpu.SMEM),
            pl.BlockSpec(memory_space=pltpu.VMEM),
        ],
        out_specs=pl.BlockSpec(memory_space=pltpu.VMEM),
    )(scal, r_interval)
    bd, lu, ub, ib = _sc_gather(uidx, iidx, emb_user,
                                emb_item[:SPLIT], emb_item[SPLIT:],
                                user_lamb_w.reshape(-1, 16),
                                user_bias_w.reshape(-1, 16),
                                item_bias_w.reshape(-1, 16))
    out = pl.pallas_call(
        _tc_final_body,
        out_shape=jax.ShapeDtypeStruct((BATCH,), jnp.float32),
        in_specs=[
            pl.BlockSpec(memory_space=pltpu.SMEM),
            pl.BlockSpec(memory_space=pltpu.VMEM),
            pl.BlockSpec(memory_space=pltpu.VMEM),
            pl.BlockSpec(memory_space=pltpu.VMEM),
            pl.BlockSpec(memory_space=pltpu.VMEM),
            pl.BlockSpec(memory_space=pltpu.VMEM),
        ],
        out_specs=pl.BlockSpec(memory_space=pltpu.VMEM),
    )(scal, bd, bl, lu, ub, ib)
    return out


# speedup vs baseline: 1.4482x; 1.4482x over previous
import jax
import jax.numpy as jnp
from jax import lax
from jax.experimental import pallas as pl
from jax.experimental.pallas import tpu as pltpu
from jax.experimental.pallas import tpu_sc as plsc

BATCH = 16384
EMB_D = 64
NC = 2
NS = 16
NW = NC * NS
BPW = BATCH // NW
CHUNK = 128
NCHUNK = BPW // CHUNK
NGRP = BPW // 16


def _sc_body(eu, ei, lw, ubw, ibw, uidx, iidx,
             bd_o, lu_o, ub_o, ib_o,
             idx_u, idx_i, bidx_u, bidx_i,
             urows, irows, ublk_l, ublk_b, iblk_b,
             luv, ubv, ibv, bdv, sem):
    c = lax.axis_index("c")
    s = lax.axis_index("s")
    wid = s * NC + c
    base = wid * BPW
    rbase = wid * NCHUNK
    pltpu.sync_copy(uidx.at[pl.ds(rbase, NCHUNK)], idx_u)
    pltpu.sync_copy(iidx.at[pl.ds(rbase, NCHUNK)], idx_i)
    for j in range(NCHUNK):
        for k in range(CHUNK // 16):
            sl = pl.ds(k * 16, 16)
            bidx_u[j, sl] = lax.shift_right_logical(idx_u[j, sl], 4)
            bidx_i[j, sl] = lax.shift_right_logical(idx_i[j, sl], 4)
    cps = []
    for j in range(NCHUNK):
        dst = pl.ds(j * CHUNK, CHUNK)
        cps.append(pltpu.async_copy(eu.at[idx_u.at[j]], urows.at[dst], sem))
        cps.append(pltpu.async_copy(ei.at[idx_i.at[j]], irows.at[dst], sem))
        cps.append(pltpu.async_copy(lw.at[bidx_u.at[j]], ublk_l.at[dst], sem))
        cps.append(pltpu.async_copy(ubw.at[bidx_u.at[j]], ublk_b.at[dst], sem))
        cps.append(pltpu.async_copy(ibw.at[bidx_i.at[j]], iblk_b.at[dst], sem))
    for cp in cps:
        cp.wait()
    iota = lax.iota(jnp.int32, 16)
    fifteen = jnp.full((16,), 15, jnp.int32)
    for g in range(NGRP):
        rows = iota + (g * 16)
        sl = pl.ds((g % 8) * 16, 16)
        offs_u = jnp.bitwise_and(idx_u[g // 8, sl], fifteen)
        offs_i = jnp.bitwise_and(idx_i[g // 8, sl], fifteen)
        out_sl = pl.ds(g * 16, 16)
        luv[out_sl] = plsc.load_gather(ublk_l, [rows, offs_u])
        ubv[out_sl] = plsc.load_gather(ublk_b, [rows, offs_u])
        ibv[out_sl] = plsc.load_gather(iblk_b, [rows, offs_i])

        def dcol(dd, acc, rows=rows):
            cols = jnp.broadcast_to(dd, (16,))
            u = plsc.load_gather(urows, [rows, cols])
            it = plsc.load_gather(irows, [rows, cols])
            return acc + jnp.abs(it - u)

        acc = lax.fori_loop(0, EMB_D, dcol, jnp.zeros((16,), jnp.float32))
        bdv[out_sl] = acc
    pltpu.sync_copy(bdv, bd_o.at[pl.ds(base, BPW)])
    pltpu.sync_copy(luv, lu_o.at[pl.ds(base, BPW)])
    pltpu.sync_copy(ubv, ub_o.at[pl.ds(base, BPW)])
    pltpu.sync_copy(ibv, ib_o.at[pl.ds(base, BPW)])


def _sc_gather(uidx2d, iidx2d, eu, ei, lw16, ubw16, ibw16):
    mesh = plsc.VectorSubcoreMesh(core_axis_name="c", subcore_axis_name="s")
    f = pl.kernel(
        _sc_body,
        out_type=[
            jax.ShapeDtypeStruct((BATCH,), jnp.float32),
            jax.ShapeDtypeStruct((BATCH,), jnp.float32),
            jax.ShapeDtypeStruct((BATCH,), jnp.float32),
            jax.ShapeDtypeStruct((BATCH,), jnp.float32),
        ],
        mesh=mesh,
        compiler_params=pltpu.CompilerParams(
            needs_layout_passes=False, use_tc_tiling_on_sc=False),
        scratch_types=[
            pltpu.VMEM((NCHUNK, CHUNK), jnp.int32),
            pltpu.VMEM((NCHUNK, CHUNK), jnp.int32),
            pltpu.VMEM((NCHUNK, CHUNK), jnp.int32),
            pltpu.VMEM((NCHUNK, CHUNK), jnp.int32),
            pltpu.VMEM((BPW, EMB_D), jnp.float32),
            pltpu.VMEM((BPW, EMB_D), jnp.float32),
            pltpu.VMEM((BPW, 16), jnp.float32),
            pltpu.VMEM((BPW, 16), jnp.float32),
            pltpu.VMEM((BPW, 16), jnp.float32),
            pltpu.VMEM((BPW,), jnp.float32),
            pltpu.VMEM((BPW,), jnp.float32),
            pltpu.VMEM((BPW,), jnp.float32),
            pltpu.VMEM((BPW,), jnp.float32),
            pltpu.SemaphoreType.DMA,
        ],
    )
    return f(eu, ei, lw16, ubw16, ibw16, uidx2d, iidx2d)


def _tc_decay_body(scal, r_ref, bl_ref):
    cc = scal[0]
    r = r_ref[...]
    mask = (r > 0.0).astype(jnp.float32)
    dt = r * mask + cc
    bl_ref[...] = jnp.sum(lax.rsqrt(dt) * mask, axis=1)


def _tc_final_body(scal, bd_ref, bl_ref, lu_ref, ub_ref, ib_ref, o_ref):
    gl = scal[1]
    alpha = scal[2]
    beta = scal[3]
    gamma = scal[4]
    bd = bd_ref[...]
    lamb = gl + jnp.clip(lu_ref[...], 0.1, 10.0)
    act = jnp.minimum(bl_ref[...] * lamb, bd)
    dist = bd - act
    logit = alpha * dist + beta * dist * dist + gamma + ub_ref[...] + ib_ref[...]
    o_ref[...] = 1.0 / (1.0 + jnp.exp(-logit))


def kernel(user_indices, item_indices, r_interval, emb_user, emb_item,
           user_lamb_w, user_bias_w, item_bias_w, global_lamb, alpha, beta,
           gamma, cutoff):
    uidx = user_indices.astype(jnp.int32).reshape(BATCH // 128, 128)
    iidx = item_indices.astype(jnp.int32).reshape(BATCH // 128, 128)
    cc = jnp.clip(cutoff, 0.1, 100.0)
    gl = jnp.clip(global_lamb, 0.01, 10.0)
    scal = jnp.stack([cc, gl, alpha, beta, gamma]).astype(jnp.float32)
    bl = pl.pallas_call(
        _tc_decay_body,
        out_shape=jax.ShapeDtypeStruct((BATCH,), jnp.float32),
        in_specs=[
            pl.BlockSpec(memory_space=pltpu.SMEM),
            pl.BlockSpec(memory_space=pltpu.VMEM),
        ],
        out_specs=pl.BlockSpec(memory_space=pltpu.VMEM),
    )(scal, r_interval)
    bd, lu, ub, ib = _sc_gather(uidx, iidx, emb_user, emb_item,
                                user_lamb_w.reshape(-1, 16),
                                user_bias_w.reshape(-1, 16),
                                item_bias_w.reshape(-1, 16))
    out = pl.pallas_call(
        _tc_final_body,
        out_shape=jax.ShapeDtypeStruct((BATCH,), jnp.float32),
        in_specs=[
            pl.BlockSpec(memory_space=pltpu.SMEM),
            pl.BlockSpec(memory_space=pltpu.VMEM),
            pl.BlockSpec(memory_space=pltpu.VMEM),
            pl.BlockSpec(memory_space=pltpu.VMEM),
            pl.BlockSpec(memory_space=pltpu.VMEM),
            pl.BlockSpec(memory_space=pltpu.VMEM),
        ],
        out_specs=pl.BlockSpec(memory_space=pltpu.VMEM),
    )(scal, bd, bl, lu, ub, ib)
    return out
